# baseline (speedup 1.0000x reference)
import functools

import jax
import jax.numpy as jnp
from jax import lax
from jax.experimental import pallas as pl
from jax.experimental.pallas import tpu as pltpu
from jax.experimental.pallas import tpu_sc as plsc

VOCAB = 1000000
EMB = 64
BATCH = 4096
HIST = 200

NC = 2
NS = 16
NW = NC * NS
ROWS_W = BATCH // NW
C0 = 104
C1 = HIST - C0
NBUF = 4
NGRP = ROWS_W // 2


def _emb_kernel(x_hbm, w_hbm, emb_hbm, mask_hbm,
                idx_vm, mask_vm, r0, r1, r2, r3, s0, s1, s2, s3):
    rbufs = (r0, r1, r2, r3)
    sems = (s0, s1, s2, s3)
    wid = lax.axis_index("s") * NC + lax.axis_index("c")
    row0 = wid * ROWS_W

    pltpu.sync_copy(x_hbm.at[pl.ds(row0, ROWS_W)], idx_vm)

    def chunk(g, k):
        r = 2 * g + k // 2
        if k % 2 == 0:
            return r, 0, C0
        return r, C0, C1

    def start(g, k):
        r, off, n = chunk(g, k)
        dst = rbufs[k] if n == C0 else rbufs[k].at[pl.ds(0, C1)]
        pltpu.make_async_copy(
            w_hbm.at[idx_vm.at[r, pl.ds(off, n)]], dst, sems[k]).start()

    def finish(g, k):
        r, off, n = chunk(g, k)
        dst = rbufs[k] if n == C0 else rbufs[k].at[pl.ds(0, C1)]
        pltpu.make_async_copy(
            w_hbm.at[idx_vm.at[r, pl.ds(off, n)]], dst, sems[k]).wait()
        pltpu.sync_copy(dst, emb_hbm.at[row0 + r, pl.ds(off, n)])

    for k in range(NBUF):
        start(0, k)

    offs = tuple(16 * t for t in range(12)) + (HIST - 16,)

    def mask_row(r, carry):
        for off in offs:
            v = idx_vm[r, pl.ds(off, 16)]
            mask_vm[r, pl.ds(off, 16)] = jnp.where(
                v != 0, jnp.float32(1.0), jnp.float32(0.0))
        return carry
    lax.fori_loop(0, ROWS_W, mask_row, 0)
    pltpu.sync_copy(mask_vm, mask_hbm.at[pl.ds(row0, ROWS_W)])

    def main_body(g, carry):
        for k in range(NBUF):
            finish(g, k)
            start(g + 1, k)
        return carry
    lax.fori_loop(0, NGRP - 1, main_body, 0)

    for k in range(NBUF):
        finish(NGRP - 1, k)


def kernel(x, W):
    kfn = functools.partial(
        pl.kernel,
        out_type=[
            jax.ShapeDtypeStruct((BATCH, HIST, EMB), jnp.float32),
            jax.ShapeDtypeStruct((BATCH, HIST), jnp.float32),
        ],
        mesh=plsc.VectorSubcoreMesh(core_axis_name="c", subcore_axis_name="s"),
        compiler_params=pltpu.CompilerParams(use_tc_tiling_on_sc=False),
        scratch_types=[
            pltpu.VMEM((ROWS_W, HIST), jnp.int32),
            pltpu.VMEM((ROWS_W, HIST), jnp.float32),
            pltpu.VMEM((C0, EMB), jnp.float32),
            pltpu.VMEM((C0, EMB), jnp.float32),
            pltpu.VMEM((C0, EMB), jnp.float32),
            pltpu.VMEM((C0, EMB), jnp.float32),
            pltpu.SemaphoreType.DMA,
            pltpu.SemaphoreType.DMA,
            pltpu.SemaphoreType.DMA,
            pltpu.SemaphoreType.DMA,
        ],
    )(_emb_kernel)
    emb, mask = kfn(x.astype(jnp.int32), W)
    return (emb, mask)
